# baseline (speedup 1.0000x reference)
import functools

import jax
import jax.numpy as jnp
from jax import lax
from jax.experimental import pallas as pl
from jax.experimental.pallas import tpu as pltpu
from jax.experimental.pallas import tpu_sc as plsc

_INFO = plsc.get_sparse_core_info()
_NC, _NS, _L = _INFO.num_cores, _INFO.num_subcores, _INFO.num_lanes
_NW = _NC * _NS

_BATCH = 16384
_D = 128
_CHUNK = 512
_K = _BATCH // (_NW * _CHUNK)


def _make_gather():
    mesh = plsc.VectorSubcoreMesh(core_axis_name="c", subcore_axis_name="s")

    @functools.partial(
        pl.kernel,
        mesh=mesh,
        out_type=jax.ShapeDtypeStruct((_NW, 2, _CHUNK // 2, _D), jnp.float32),
        compiler_params=pltpu.CompilerParams(use_tc_tiling_on_sc=False),
        scratch_types=[
            pltpu.VMEM((1, _CHUNK // 2), jnp.int32),
            pltpu.VMEM((1, _CHUNK // 2), jnp.int32),
            pltpu.VMEM((2, _CHUNK // 2, _D), jnp.float32),
            pltpu.SemaphoreType.DMA,
            pltpu.SemaphoreType.DMA,
        ],
    )
    def gather_kernel(idxa_hbm, idxb_hbm, table_hbm, out_hbm,
                      iva, ivb, rows_v, gsem, wsem):
        wid = lax.axis_index("s") * _NC + lax.axis_index("c")
        pltpu.sync_copy(idxa_hbm.at[wid], iva)
        pltpu.sync_copy(idxb_hbm.at[wid], ivb)
        ga = pltpu.async_copy(table_hbm.at[iva.at[0]], rows_v.at[0], gsem)
        gb = pltpu.async_copy(table_hbm.at[ivb.at[0]], rows_v.at[1], gsem)
        ga.wait()
        wa = pltpu.async_copy(rows_v.at[0], out_hbm.at[wid, 0], wsem)
        gb.wait()
        wb = pltpu.async_copy(rows_v.at[1], out_hbm.at[wid, 1], wsem)
        wa.wait()
        wb.wait()

    return gather_kernel


_GATHER = _make_gather()


@jax.jit
def kernel(y, codebook):
    idx = y.astype(jnp.int32).reshape(_NW, 2, _CHUNK // 2)
    idxa = idx[:, 0].reshape(_NW, 1, _CHUNK // 2)
    idxb = idx[:, 1].reshape(_NW, 1, _CHUNK // 2)
    out = _GATHER(idxa, idxb, codebook)
    return out.reshape(_BATCH, _D)


# speedup vs baseline: 1.0610x; 1.0610x over previous
import functools

import jax
import jax.numpy as jnp
from jax import lax
from jax.experimental import pallas as pl
from jax.experimental.pallas import tpu as pltpu
from jax.experimental.pallas import tpu_sc as plsc

_INFO = plsc.get_sparse_core_info()
_NC, _NS, _L = _INFO.num_cores, _INFO.num_subcores, _INFO.num_lanes
_NW = _NC * _NS

_BATCH = 16384
_D = 128
_CHUNK = 512
_K = _BATCH // (_NW * _CHUNK)


def _make_gather():
    mesh = plsc.VectorSubcoreMesh(core_axis_name="c", subcore_axis_name="s")

    @functools.partial(
        pl.kernel,
        mesh=mesh,
        out_type=jax.ShapeDtypeStruct((_NW, _K, _CHUNK, _D), jnp.float32),
        compiler_params=pltpu.CompilerParams(use_tc_tiling_on_sc=False),
        scratch_types=[
            pltpu.VMEM((_K, _CHUNK), jnp.int32),
            pltpu.VMEM((_K, _CHUNK, _D), jnp.float32),
            pltpu.SemaphoreType.DMA,
        ],
    )
    def gather_kernel(idx_hbm, table_hbm, out_hbm, idx_v, rows_v, sem):
        wid = lax.axis_index("s") * _NC + lax.axis_index("c")
        pltpu.sync_copy(idx_hbm.at[wid], idx_v)
        copies = [
            pltpu.async_copy(table_hbm.at[idx_v.at[j]], rows_v.at[j], sem)
            for j in range(_K)
        ]
        for c in copies:
            c.wait()
        pltpu.sync_copy(rows_v, out_hbm.at[wid])

    return gather_kernel


_GATHER = _make_gather()


@jax.jit
def kernel(y, codebook):
    idx = y.astype(jnp.int32).reshape(_NW, _K, _CHUNK)
    out = _GATHER(idx, codebook)
    return out.reshape(_BATCH, _D)
